# baseline (speedup 1.0000x reference)
import functools

import jax
import jax.numpy as jnp
from jax.experimental import pallas as pl
from jax.experimental.pallas import tpu as pltpu

B, S, D, E, TOPK = 4, 8192, 768, 64, 2
BS = B * S
BM = 512


_A = (0.999999328, -0.33326375, 0.198798757, -0.134804157,
      0.0837416936, -0.0368987135, 0.00782550109)
_Q = (1.0000002, 0.499979855, -0.124654664, 0.0602089666,
      -0.0314031491, 0.012638648, -0.00255643702)
_HALF_PI = 1.5707963267948966
_PI = 3.141592653589793
_IMIN = -(2**31)


def _poly7(z, z2, z4, c):
    e01 = c[1] * z + c[0]
    e23 = c[3] * z + c[2]
    e45 = c[5] * z + c[4]
    h0 = e23 * z2 + e01
    h1 = c[6] * z2 + e45
    return h1 * z4 + h0


def _key_to_float(k):
    sb = k ^ ((k >> 31) & jnp.int32(0x7FFFFFFF))
    return pltpu.bitcast(sb, jnp.float32)


def _gating_kernel(xr_ref, xi_ref, wa_ref, wp_ref, b_ref, probs_ref, idx_ref):
    xr = xr_ref[...]
    xi = xi_ref[...]
    ax = jnp.abs(xr)
    ay = jnp.abs(xi)
    mx = jnp.maximum(ax, ay)
    mn = jnp.minimum(ax, ay)
    t = mn / jnp.maximum(mx, 1e-35)
    z = t * t
    z2 = z * z
    z4 = z2 * z2
    r = t * _poly7(z, z2, z4, _A)
    r = jnp.where(ay > ax, _HALF_PI - r, r)
    r = jnp.where(xr < 0, _PI - r, r)
    phase = pltpu.bitcast(
        pltpu.bitcast(r, jnp.int32)
        ^ (pltpu.bitcast(xi, jnp.int32) & jnp.int32(-(2**31))),
        jnp.float32,
    )
    amp = mx * _poly7(z, z2, z4, _Q)

    scores = (
        jnp.dot(amp, wa_ref[...], preferred_element_type=jnp.float32)
        + jnp.dot(phase, wp_ref[...], preferred_element_type=jnp.float32)
        + b_ref[...]
    )

    sb = pltpu.bitcast(scores, jnp.int32)
    k = sb ^ ((sb >> 31) & jnp.int32(0x7FFFFFFF))
    revlane = 63 - jax.lax.broadcasted_iota(jnp.int32, scores.shape, 1)
    kp = (k & jnp.int32(-64)) | revlane
    m1 = jnp.max(kp, axis=-1, keepdims=True)
    masked = jnp.where(kp == m1, _IMIN, kp)
    m2 = jnp.max(masked, axis=-1, keepdims=True)

    i1 = 63 - (m1 & 63)
    i2 = 63 - (m2 & 63)
    s1 = _key_to_float(m1 & jnp.int32(-64))
    s2 = _key_to_float(m2 & jnp.int32(-64))
    e = jnp.exp(s2 - s1)
    p0 = 1.0 / (1.0 + e)
    probs_ref[:, 0:1] = p0
    probs_ref[:, 1:2] = 1.0 - p0
    idx_ref[:, 0:1] = i1
    idx_ref[:, 1:2] = i2


@jax.jit
def kernel(x_real, x_imag, W, b):
    xr = x_real.reshape(BS, D)
    xi = x_imag.reshape(BS, D)
    wa = W[:D]
    wp = W[D:]
    b2 = b.reshape(1, E)

    grid = (BS // BM,)
    probs, idx = pl.pallas_call(
        _gating_kernel,
        grid=grid,
        in_specs=[
            pl.BlockSpec((BM, D), lambda i: (i, 0)),
            pl.BlockSpec((BM, D), lambda i: (i, 0)),
            pl.BlockSpec((D, E), lambda i: (0, 0)),
            pl.BlockSpec((D, E), lambda i: (0, 0)),
            pl.BlockSpec((1, E), lambda i: (0, 0)),
        ],
        out_specs=[
            pl.BlockSpec((BM, TOPK), lambda i: (i, 0)),
            pl.BlockSpec((BM, TOPK), lambda i: (i, 0)),
        ],
        out_shape=[
            jax.ShapeDtypeStruct((BS, TOPK), jnp.float32),
            jax.ShapeDtypeStruct((BS, TOPK), jnp.int32),
        ],
        compiler_params=pltpu.CompilerParams(
            dimension_semantics=("arbitrary",),
        ),
    )(xr, xi, wa, wp, b2)

    return probs.reshape(B, S, TOPK), idx.reshape(B, S, TOPK)


# speedup vs baseline: 1.1047x; 1.1047x over previous
import functools

import jax
import jax.numpy as jnp
from jax.experimental import pallas as pl
from jax.experimental.pallas import tpu as pltpu

B, S, D, E, TOPK = 4, 8192, 768, 64, 2
BS = B * S
BM = 512


_A = (0.999995508, -0.33298865, 0.195589143, -0.121109628,
      0.0573306763, -0.0134222103)
_Q = (1.00000144, 0.499889985, -0.123606147, 0.0557352338,
      -0.0227751901, 0.00496933691)
_HALF_PI = 1.5707963267948966
_PI = 3.141592653589793
_IMIN = -(2**31)


def _poly(z, coeffs):
    p = coeffs[-1]
    for c in reversed(coeffs[:-1]):
        p = p * z + c
    return p


def _key_to_float(k):
    sb = k ^ ((k >> 31) & jnp.int32(0x7FFFFFFF))
    return pltpu.bitcast(sb, jnp.float32)


def _gating_kernel(xr_ref, xi_ref, wa_ref, wp_ref, b_ref, probs_ref, idx_ref):
    xr = xr_ref[...]
    xi = xi_ref[...]
    ax = jnp.abs(xr)
    ay = jnp.abs(xi)
    mx = jnp.maximum(ax, ay)
    mn = jnp.minimum(ax, ay)
    t = mn / jnp.maximum(mx, 1e-35)
    z = t * t
    r = t * _poly(z, _A)
    r = jnp.where(ay > ax, _HALF_PI - r, r)
    r = jnp.where(xr < 0, _PI - r, r)
    phase = pltpu.bitcast(
        pltpu.bitcast(r, jnp.int32)
        ^ (pltpu.bitcast(xi, jnp.int32) & jnp.int32(-(2**31))),
        jnp.float32,
    )
    amp = mx * _poly(z, _Q)

    scores = (
        jnp.dot(amp, wa_ref[...], preferred_element_type=jnp.float32)
        + jnp.dot(phase, wp_ref[...], preferred_element_type=jnp.float32)
        + b_ref[...]
    )

    sb = pltpu.bitcast(scores, jnp.int32)
    k = sb ^ ((sb >> 31) & jnp.int32(0x7FFFFFFF))
    revlane = 63 - jax.lax.broadcasted_iota(jnp.int32, scores.shape, 1)
    kp = (k & jnp.int32(-64)) | revlane
    m1 = jnp.max(kp, axis=-1, keepdims=True)
    masked = jnp.where(kp == m1, _IMIN, kp)
    m2 = jnp.max(masked, axis=-1, keepdims=True)

    i1 = 63 - (m1 & 63)
    i2 = 63 - (m2 & 63)
    s1 = _key_to_float(m1 & jnp.int32(-64))
    s2 = _key_to_float(m2 & jnp.int32(-64))
    e = jnp.exp(s2 - s1)
    p0 = 1.0 / (1.0 + e)
    probs_ref[:, 0:1] = p0
    probs_ref[:, 1:2] = 1.0 - p0
    idx_ref[:, 0:1] = i1
    idx_ref[:, 1:2] = i2


@jax.jit
def kernel(x_real, x_imag, W, b):
    xr = x_real.reshape(BS, D)
    xi = x_imag.reshape(BS, D)
    wa = W[:D]
    wp = W[D:]
    b2 = b.reshape(1, E)

    grid = (BS // BM,)
    probs, idx = pl.pallas_call(
        _gating_kernel,
        grid=grid,
        in_specs=[
            pl.BlockSpec((BM, D), lambda i: (i, 0)),
            pl.BlockSpec((BM, D), lambda i: (i, 0)),
            pl.BlockSpec((D, E), lambda i: (0, 0)),
            pl.BlockSpec((D, E), lambda i: (0, 0)),
            pl.BlockSpec((1, E), lambda i: (0, 0)),
        ],
        out_specs=[
            pl.BlockSpec((BM, TOPK), lambda i: (i, 0)),
            pl.BlockSpec((BM, TOPK), lambda i: (i, 0)),
        ],
        out_shape=[
            jax.ShapeDtypeStruct((BS, TOPK), jnp.float32),
            jax.ShapeDtypeStruct((BS, TOPK), jnp.int32),
        ],
        compiler_params=pltpu.CompilerParams(
            dimension_semantics=("arbitrary",),
        ),
    )(xr, xi, wa, wp, b2)

    return probs.reshape(B, S, TOPK), idx.reshape(B, S, TOPK)


# speedup vs baseline: 1.1357x; 1.0281x over previous
import functools

import jax
import jax.numpy as jnp
from jax.experimental import pallas as pl
from jax.experimental.pallas import tpu as pltpu

B, S, D, E, TOPK = 4, 8192, 768, 64, 2
BS = B * S
BM = 512


_A = (0.999995508, -0.33298865, 0.195589143, -0.121109628,
      0.0573306763, -0.0134222103)
_Q = (1.00000144, 0.499889985, -0.123606147, 0.0557352338,
      -0.0227751901, 0.00496933691)
_HALF_PI = 1.5707963267948966
_PI = 3.141592653589793
_IMIN = -(2**31)


def _poly(z, coeffs):
    p = coeffs[-1]
    for c in reversed(coeffs[:-1]):
        p = p * z + c
    return p


def _key_to_float(k):
    sb = k ^ ((k >> 31) & jnp.int32(0x7FFFFFFF))
    return pltpu.bitcast(sb, jnp.float32)


def _gating_kernel(xr_ref, xi_ref, wa_ref, wp_ref, b_ref, probs_ref, idx_ref):
    xr = xr_ref[...]
    xi = xi_ref[...]
    ax = jnp.abs(xr)
    ay = jnp.abs(xi)
    mx = jnp.maximum(ax, ay)
    mn = jnp.minimum(ax, ay)
    t = mn / jnp.maximum(mx, 1e-35)
    z = t * t
    r = t * _poly(z, _A)
    r = jnp.where(ay > ax, _HALF_PI - r, r)
    r = jnp.where(xr < 0, _PI - r, r)
    phase = pltpu.bitcast(
        pltpu.bitcast(r, jnp.int32)
        ^ (pltpu.bitcast(xi, jnp.int32) & jnp.int32(-(2**31))),
        jnp.float32,
    )
    amp = mx * _poly(z, _Q)

    scores = (
        jnp.dot(amp, wa_ref[...], preferred_element_type=jnp.float32)
        + jnp.dot(phase, wp_ref[...], preferred_element_type=jnp.float32)
        + b_ref[...]
    )

    rlf = (63 - jax.lax.broadcasted_iota(jnp.int32, scores.shape, 1)).astype(
        jnp.float32)
    m1 = jnp.max(scores, axis=-1, keepdims=True)
    rl1 = jnp.max(jnp.where(scores == m1, rlf, -1.0), axis=-1, keepdims=True)
    masked = jnp.where(rlf == rl1, -jnp.inf, scores)
    m2 = jnp.max(masked, axis=-1, keepdims=True)
    rl2 = jnp.max(jnp.where(masked == m2, rlf, -1.0), axis=-1, keepdims=True)

    i1 = 63 - rl1.astype(jnp.int32)
    i2 = 63 - rl2.astype(jnp.int32)
    e = jnp.exp(m2 - m1)
    p0 = 1.0 / (1.0 + e)
    probs_ref[:, 0:1] = p0
    probs_ref[:, 1:2] = 1.0 - p0
    idx_ref[:, 0:1] = i1
    idx_ref[:, 1:2] = i2


@jax.jit
def kernel(x_real, x_imag, W, b):
    xr = x_real.reshape(BS, D)
    xi = x_imag.reshape(BS, D)
    wa = W[:D]
    wp = W[D:]
    b2 = b.reshape(1, E)

    grid = (BS // BM,)
    probs, idx = pl.pallas_call(
        _gating_kernel,
        grid=grid,
        in_specs=[
            pl.BlockSpec((BM, D), lambda i: (i, 0)),
            pl.BlockSpec((BM, D), lambda i: (i, 0)),
            pl.BlockSpec((D, E), lambda i: (0, 0)),
            pl.BlockSpec((D, E), lambda i: (0, 0)),
            pl.BlockSpec((1, E), lambda i: (0, 0)),
        ],
        out_specs=[
            pl.BlockSpec((BM, TOPK), lambda i: (i, 0)),
            pl.BlockSpec((BM, TOPK), lambda i: (i, 0)),
        ],
        out_shape=[
            jax.ShapeDtypeStruct((BS, TOPK), jnp.float32),
            jax.ShapeDtypeStruct((BS, TOPK), jnp.int32),
        ],
        compiler_params=pltpu.CompilerParams(
            dimension_semantics=("arbitrary",),
        ),
    )(xr, xi, wa, wp, b2)

    return probs.reshape(B, S, TOPK), idx.reshape(B, S, TOPK)
